# baseline (speedup 1.0000x reference)
import functools

import jax
import jax.numpy as jnp
import numpy as np
from jax import lax
from jax.experimental import pallas as pl
from jax.experimental.pallas import tpu as pltpu
from jax.experimental.pallas import tpu_sc as plsc

_PROB = 0.5
_NUM_PREFIX = 1
_B, _T, _D = 128, 577, 768
_NP = _T - _NUM_PREFIX
_NK = max(1, int(_NP * (1.0 - _PROB)))
_ROWS = _NUM_PREFIX + _NK
_NW = 32
_TPW = _ROWS // _NW
_NBUF = 4
_CHK = _B // _NBUF
_HB = _B // 2


@functools.lru_cache(maxsize=1)
def _gather_indices() -> np.ndarray:
    with jax.ensure_compile_time_eval():
        rand = jax.random.normal(jax.random.key(42), (_B, _NP), dtype=jnp.float32)
        order = jnp.argsort(rand, axis=-1)
        keep = jnp.sort(order[:, :_NK], axis=-1) + _NUM_PREFIX
        full = jnp.concatenate(
            [jnp.zeros((_B, _NUM_PREFIX), keep.dtype), keep], axis=1)
    rows = np.asarray(full).astype(np.int32)
    flat = rows.T * _B + np.arange(_B, dtype=np.int32)[None, :]
    return flat.reshape(_ROWS, 1, _B)


def _sc_gather(x2, idx3):
    mesh = plsc.VectorSubcoreMesh(core_axis_name="c", subcore_axis_name="s")

    @functools.partial(
        pl.kernel,
        mesh=mesh,
        out_type=jax.ShapeDtypeStruct((_ROWS * _B, _D), jnp.float32),
        scratch_types=[
            pltpu.VMEM((_TPW + 1, 1, _B), jnp.int32),
            *[pltpu.VMEM((_CHK, _D), jnp.float32) for _ in range(_NBUF)],
            *[pltpu.SemaphoreType.DMA for _ in range(2 * _NBUF)],
        ],
    )
    def gather_rows(x_hbm, idx_hbm, out_hbm, idx_v, *rest):
        bufs = rest[:_NBUF]
        gsems = rest[_NBUF:2 * _NBUF]
        wsems = rest[2 * _NBUF:]
        wid = lax.axis_index("s") * 2 + lax.axis_index("c")

        base_tok = wid * _TPW
        pltpu.sync_copy(idx_hbm.at[pl.ds(base_tok, _TPW)],
                        idx_v.at[pl.ds(0, _TPW)])
        @pl.when(wid >= _NW - 2)
        def _():
            pltpu.sync_copy(idx_hbm.at[_ROWS - 1], idx_v.at[_TPW])

        jobs = [(k, q) for k in range(_TPW) for q in range(_NBUF)]
        gds = [None] * len(jobs)
        last_write = [None] * _NBUF

        def row_off(k, q):
            return pl.multiple_of((base_tok + k) * _B + _CHK * q, _CHK)

        def start_gather(j):
            k, q = jobs[j]
            tag = j % _NBUF
            if last_write[tag] is not None:
                last_write[tag].wait()
                last_write[tag] = None
            gds[j] = pltpu.async_copy(
                x_hbm.at[idx_v.at[k].at[0].at[pl.ds(_CHK * q, _CHK)]],
                bufs[tag], gsems[tag])

        nj = len(jobs)
        for j in range(_NBUF - 1):
            start_gather(j)
        for j in range(nj):
            if j + _NBUF - 1 < nj:
                start_gather(j + _NBUF - 1)
            gds[j].wait()
            k, q = jobs[j]
            tag = j % _NBUF
            last_write[tag] = pltpu.async_copy(
                bufs[tag], out_hbm.at[pl.ds(row_off(k, q), _CHK)], wsems[tag])
        for tag in range(_NBUF):
            if last_write[tag] is not None:
                last_write[tag].wait()
                last_write[tag] = None

        for w, h in ((_NW - 2, 0), (_NW - 1, 1)):
            @pl.when(wid == w)
            def _(h=h):
                for p in range(_NBUF // 2):
                    q = h * (_NBUF // 2) + p
                    pltpu.async_copy(
                        x_hbm.at[idx_v.at[_TPW].at[0].at[pl.ds(_CHK * q, _CHK)]],
                        bufs[p], gsems[p]).wait()
                    pltpu.sync_copy(
                        bufs[p],
                        out_hbm.at[pl.ds((_ROWS - 1) * _B + _CHK * q, _CHK)])

    return gather_rows(x2, idx3)


def kernel(inputs):
    x = inputs
    x2 = jnp.transpose(x, (1, 0, 2)).reshape(_T * _B, _D)
    idx3 = jnp.asarray(_gather_indices())
    out2 = _sc_gather(x2, idx3)
    return jnp.transpose(out2.reshape(_ROWS, _B, _D), (1, 0, 2))


# speedup vs baseline: 1.0182x; 1.0182x over previous
import functools

import jax
import jax.numpy as jnp
import numpy as np
from jax import lax
from jax.experimental import pallas as pl
from jax.experimental.pallas import tpu as pltpu
from jax.experimental.pallas import tpu_sc as plsc

_PROB = 0.5
_NUM_PREFIX = 1
_B, _T, _D = 128, 577, 768
_NP = _T - _NUM_PREFIX
_NK = max(1, int(_NP * (1.0 - _PROB)))
_ROWS = _NUM_PREFIX + _NK
_NW = 32
_TPW = _ROWS // _NW
_NBUF = 2
_CHK = _B // _NBUF
_HB = _B // 2


@functools.lru_cache(maxsize=1)
def _gather_indices() -> np.ndarray:
    with jax.ensure_compile_time_eval():
        rand = jax.random.normal(jax.random.key(42), (_B, _NP), dtype=jnp.float32)
        order = jnp.argsort(rand, axis=-1)
        keep = jnp.sort(order[:, :_NK], axis=-1) + _NUM_PREFIX
        full = jnp.concatenate(
            [jnp.zeros((_B, _NUM_PREFIX), keep.dtype), keep], axis=1)
    rows = np.asarray(full).astype(np.int32)
    flat = rows.T * _B + np.arange(_B, dtype=np.int32)[None, :]
    return flat.reshape(_ROWS, 1, _B)


def _sc_gather(x2, idx3):
    mesh = plsc.VectorSubcoreMesh(core_axis_name="c", subcore_axis_name="s")

    @functools.partial(
        pl.kernel,
        mesh=mesh,
        out_type=jax.ShapeDtypeStruct((_ROWS * _B, _D), jnp.float32),
        scratch_types=[
            pltpu.VMEM((_TPW + 1, 1, _B), jnp.int32),
            *[pltpu.VMEM((_CHK, _D), jnp.float32) for _ in range(_NBUF)],
            *[pltpu.SemaphoreType.DMA for _ in range(2 * _NBUF)],
        ],
    )
    def gather_rows(x_hbm, idx_hbm, out_hbm, idx_v, *rest):
        bufs = rest[:_NBUF]
        gsems = rest[_NBUF:2 * _NBUF]
        wsems = rest[2 * _NBUF:]
        wid = lax.axis_index("s") * 2 + lax.axis_index("c")

        base_tok = wid * _TPW
        pltpu.sync_copy(idx_hbm.at[pl.ds(base_tok, _TPW)],
                        idx_v.at[pl.ds(0, _TPW)])
        @pl.when(wid >= _NW - 2)
        def _():
            pltpu.sync_copy(idx_hbm.at[_ROWS - 1], idx_v.at[_TPW])

        jobs = [(k, q) for k in range(_TPW) for q in range(_NBUF)]
        gds = [None] * len(jobs)
        last_write = [None] * _NBUF

        def row_off(k, q):
            return pl.multiple_of((base_tok + k) * _B + _CHK * q, _CHK)

        def start_gather(j):
            k, q = jobs[j]
            tag = j % _NBUF
            if last_write[tag] is not None:
                last_write[tag].wait()
                last_write[tag] = None
            gds[j] = pltpu.async_copy(
                x_hbm.at[idx_v.at[k].at[0].at[pl.ds(_CHK * q, _CHK)]],
                bufs[tag], gsems[tag])

        nj = len(jobs)
        for j in range(_NBUF - 1):
            start_gather(j)
        for j in range(nj):
            if j + _NBUF - 1 < nj:
                start_gather(j + _NBUF - 1)
            gds[j].wait()
            k, q = jobs[j]
            tag = j % _NBUF
            last_write[tag] = pltpu.async_copy(
                bufs[tag], out_hbm.at[pl.ds(row_off(k, q), _CHK)], wsems[tag])
        for tag in range(_NBUF):
            if last_write[tag] is not None:
                last_write[tag].wait()
                last_write[tag] = None

        for w, h in ((_NW - 2, 0), (_NW - 1, 1)):
            @pl.when(wid == w)
            def _(h=h):
                for p in range(_NBUF // 2):
                    q = h * (_NBUF // 2) + p
                    pltpu.async_copy(
                        x_hbm.at[idx_v.at[_TPW].at[0].at[pl.ds(_CHK * q, _CHK)]],
                        bufs[p], gsems[p]).wait()
                    pltpu.sync_copy(
                        bufs[p],
                        out_hbm.at[pl.ds((_ROWS - 1) * _B + _CHK * q, _CHK)])

    return gather_rows(x2, idx3)


def kernel(inputs):
    x = inputs
    x2 = jnp.transpose(x, (1, 0, 2)).reshape(_T * _B, _D)
    idx3 = jnp.asarray(_gather_indices())
    out2 = _sc_gather(x2, idx3)
    return jnp.transpose(out2.reshape(_ROWS, _B, _D), (1, 0, 2))
